# baseline (speedup 1.0000x reference)
import functools

import jax
import jax.numpy as jnp
from jax import lax
from jax.experimental import pallas as pl
from jax.experimental.pallas import tpu as pltpu
from jax.experimental.pallas import tpu_sc as plsc

_NC = 2
_NS = 16
_NW = _NC * _NS

_D = 64
_G = 16
_PF = 32
_PD = 128


@functools.cache
def _make_gather(batch, fields):
    bpw = batch // _NW
    ngrp = bpw // _G
    mesh = plsc.VectorSubcoreMesh(core_axis_name="c", subcore_axis_name="s")

    @functools.partial(
        pl.kernel,
        mesh=mesh,
        out_type=jax.ShapeDtypeStruct((batch, _PF, _PD), jnp.float32),
        scratch_types=[
            pltpu.VMEM((bpw * _PF,), jnp.int32),
            pltpu.VMEM((2, _G, fields, _PD), jnp.float32),
            pltpu.SemaphoreType.DMA,
            pltpu.SemaphoreType.DMA,
        ],
        compiler_params=pltpu.CompilerParams(use_tc_tiling_on_sc=False),
    )
    def gather_kernel(idx_hbm, table_hbm, out_hbm, idx_v, rows_v, gsem, wsem):
        wid = lax.axis_index("s") * _NC + lax.axis_index("c")
        base = wid * bpw
        pltpu.sync_copy(idx_hbm.at[pl.ds(base * _PF, bpw * _PF)], idx_v)

        def fire_group_gathers(g, buf):
            for k in range(_G):
                pltpu.async_copy(
                    table_hbm.at[idx_v.at[pl.ds((g * _G + k) * _PF,
                                                fields)]],
                    rows_v.at[buf, k], gsem)

        def wait_group_gathers():
            for _ in range(_G):
                pltpu.make_async_copy(
                    table_hbm.at[idx_v.at[pl.ds(0, fields)]],
                    rows_v.at[0, 0], gsem).wait()

        def fire_write(g, buf):
            pltpu.async_copy(
                rows_v.at[buf, :, :, pl.ds(0, _D)],
                out_hbm.at[pl.ds(base + g * _G, _G), pl.ds(0, fields),
                           pl.ds(0, _D)], wsem)

        def wait_write():
            pltpu.make_async_copy(
                rows_v.at[0, :, :, pl.ds(0, _D)],
                out_hbm.at[pl.ds(0, _G), pl.ds(0, fields), pl.ds(0, _D)],
                wsem).wait()

        fire_group_gathers(0, 0)

        def body(g, carry):
            nxt = lax.rem(g + 1, 2)

            @pl.when(g + 1 < ngrp)
            def _prefetch():
                @pl.when(g >= 1)
                def _free_buf():
                    wait_write()

                fire_group_gathers(g + 1, nxt)

            wait_group_gathers()
            fire_write(g, lax.rem(g, 2))
            return carry

        lax.fori_loop(0, ngrp, body, 0)
        wait_write()
        wait_write()

    return gather_kernel


def kernel(weight, mask, input):
    b, f = input.shape
    idx = jnp.pad(input.astype(jnp.int32), ((0, 0), (0, _PF - f)))
    wp = jnp.pad(weight, ((0, 0), (0, _PD - _D)))
    o = _make_gather(b, f)(idx.reshape(-1), wp)
    return o[:, :f, :_D]


# speedup vs baseline: 1.0276x; 1.0276x over previous
import functools

import jax
import jax.numpy as jnp
from jax import lax
from jax.experimental import pallas as pl
from jax.experimental.pallas import tpu as pltpu
from jax.experimental.pallas import tpu_sc as plsc

_NC = 2
_NS = 16
_NW = _NC * _NS

_D = 64
_G = 16
_PF = 32
_PD = 128


@functools.cache
def _make_gather(batch, fields):
    bpw = batch // _NW
    ngrp = bpw // _G
    mesh = plsc.VectorSubcoreMesh(core_axis_name="c", subcore_axis_name="s")

    @functools.partial(
        pl.kernel,
        mesh=mesh,
        out_type=jax.ShapeDtypeStruct((batch, _PF, _PD), jnp.float32),
        scratch_types=[
            pltpu.VMEM((bpw * _PF,), jnp.int32),
            pltpu.VMEM((2, _G, fields, _D), jnp.float32),
            pltpu.SemaphoreType.DMA,
            pltpu.SemaphoreType.DMA,
        ],
        compiler_params=pltpu.CompilerParams(use_tc_tiling_on_sc=False),
    )
    def gather_kernel(idx_hbm, table_hbm, out_hbm, idx_v, rows_v, gsem, wsem):
        wid = lax.axis_index("s") * _NC + lax.axis_index("c")
        base = wid * bpw
        pltpu.sync_copy(idx_hbm.at[pl.ds(base * _PF, bpw * _PF)], idx_v)

        def fire_group_gathers(g, buf):
            for k in range(_G):
                pltpu.async_copy(
                    table_hbm.at[idx_v.at[pl.ds((g * _G + k) * _PF,
                                                fields)]],
                    rows_v.at[buf, k], gsem)

        def wait_group_gathers():
            for _ in range(_G):
                pltpu.make_async_copy(
                    table_hbm.at[idx_v.at[pl.ds(0, fields)]],
                    rows_v.at[0, 0], gsem).wait()

        def fire_write(g, buf):
            pltpu.async_copy(
                rows_v.at[buf],
                out_hbm.at[pl.ds(base + g * _G, _G), pl.ds(0, fields),
                           pl.ds(0, _D)], wsem)

        def wait_write():
            pltpu.make_async_copy(
                rows_v.at[0],
                out_hbm.at[pl.ds(0, _G), pl.ds(0, fields), pl.ds(0, _D)],
                wsem).wait()

        fire_group_gathers(0, 0)

        def body(g, carry):
            nxt = lax.rem(g + 1, 2)

            @pl.when(g + 1 < ngrp)
            def _prefetch():
                @pl.when(g >= 1)
                def _free_buf():
                    wait_write()

                fire_group_gathers(g + 1, nxt)

            wait_group_gathers()
            fire_write(g, lax.rem(g, 2))
            return carry

        lax.fori_loop(0, ngrp, body, 0)
        wait_write()
        wait_write()

    return gather_kernel


def kernel(weight, mask, input):
    b, f = input.shape
    idx = jnp.pad(input.astype(jnp.int32), ((0, 0), (0, _PF - f)))
    o = _make_gather(b, f)(idx.reshape(-1), weight)
    return o[:, :f, :_D]
